# baseline (device time: 99432 ns/iter reference)
import jax
import jax.numpy as jnp
from jax import lax
from jax.experimental import pallas as pl
from jax.experimental.pallas import tpu as pltpu

N_DEV = 16
B_LOC = 2
SQ = 256
SKV = 256
HQ_LOC = 4
DH = 64
D_MODEL = 512
BLK = 64


def kernel(x, Wq, K_ext, V_ext, Wo):
    i = lax.axis_index("i")
    xb = x.astype(jnp.bfloat16)
    wq = (Wq * 0.125).astype(jnp.bfloat16)
    wo = Wo.astype(jnp.bfloat16)
    k_loc = lax.dynamic_slice_in_dim(K_ext, i * B_LOC, B_LOC, axis=0)
    v_loc = lax.dynamic_slice_in_dim(V_ext, i * B_LOC, B_LOC, axis=0)
    k_loc = jnp.transpose(k_loc, (2, 0, 1, 3)).astype(jnp.bfloat16)
    v_loc = jnp.transpose(v_loc, (2, 0, 1, 3)).astype(jnp.bfloat16)

    def body(x_ref, wq_ref, wo_ref, k_ref, v_ref, out_ref,
             wq_comm, wo_comm, wq_send, wq_recv, wo_send, wo_recv,
             x_pe, k_re, v_re, acc_ref):
        my = lax.axis_index("i")
        right = lax.rem(my + 1, N_DEV)
        left = lax.rem(my + N_DEV - 1, N_DEV)

        barrier = pltpu.get_barrier_semaphore()
        pl.semaphore_signal(barrier, inc=1, device_id=(left,),
                            device_id_type=pl.DeviceIdType.MESH)
        pl.semaphore_signal(barrier, inc=1, device_id=(right,),
                            device_id_type=pl.DeviceIdType.MESH)
        pl.semaphore_wait(barrier, 2)

        wq_comm[0] = wq_ref[...]
        wo_comm[0] = wo_ref[...]

        def mk(to_right, src, dst, ssem, rsem):
            tgt = lax.select(to_right, right, left)
            rq = pltpu.make_async_remote_copy(
                src_ref=wq_comm.at[src], dst_ref=wq_comm.at[dst],
                send_sem=wq_send.at[ssem], recv_sem=wq_recv.at[rsem],
                device_id=(tgt,), device_id_type=pl.DeviceIdType.MESH)
            ro = pltpu.make_async_remote_copy(
                src_ref=wo_comm.at[src], dst_ref=wo_comm.at[dst],
                send_sem=wo_send.at[ssem], recv_sem=wo_recv.at[rsem],
                device_id=(tgt,), device_id_type=pl.DeviceIdType.MESH)
            return rq, ro

        def start(descs):
            descs[0].start()
            descs[1].start()

        def wait(descs):
            descs[0].wait()
            descs[1].wait()

        t_one = jnp.bool_(True)
        t_zero = jnp.bool_(False)
        start(mk(t_one, 0, 1, 0, 1))
        start(mk(t_zero, 0, 9, 8, 9))

        x_pe[:, 0:64] = x_ref[:, 0:64]
        x_pe[:, 64:128] = x_ref[:, 192:256]
        x_pe[:, 128:256] = x_ref[:, 64:192]
        k_re[:, :, 0:64] = k_ref[:, :, 0:64]
        k_re[:, :, 64:128] = k_ref[:, :, 192:256]
        k_re[:, :, 128:320] = k_ref[:, :, 0:192]
        v_re[:, :, 0:64] = v_ref[:, :, 0:64]
        v_re[:, :, 64:128] = v_ref[:, :, 192:256]
        v_re[:, :, 128:320] = v_ref[:, :, 0:192]
        acc_ref[...] = jnp.zeros((B_LOC, SQ, D_MODEL), jnp.float32)

        def attn(q, k, v):
            s = lax.dot_general(q, k, (((1,), (1,)), ((), ())),
                                preferred_element_type=jnp.float32)
            u = jnp.exp(s)
            r = 1.0 / jnp.sum(u, axis=-1, keepdims=True)
            c = jnp.dot(u.astype(jnp.bfloat16), v,
                        preferred_element_type=jnp.float32)
            return (c * r).astype(jnp.bfloat16)

        def compute(slot, j):
            wq_s = wq_comm[slot]
            wo_s = wo_comm[slot]
            for b in range(B_LOC):
                q_all = jnp.dot(x_pe[b], wq_s,
                                preferred_element_type=jnp.float32)
                q_all = q_all.astype(jnp.bfloat16)
                ctxs = []
                for hh in range(HQ_LOC):
                    gh = j * HQ_LOC + hh
                    sl = slice(hh * DH, (hh + 1) * DH)
                    c_a = attn(q_all[0:128, sl], k_re[gh, b, 0:128],
                               v_re[gh, b, 0:128])
                    c_b = attn(q_all[128:256, sl], k_re[gh, b, 128:320],
                               v_re[gh, b, 128:320])
                    ctxs.append(jnp.concatenate([c_a, c_b], axis=0))
                ctx_full = jnp.concatenate(ctxs, axis=1)
                acc_ref[b] += jnp.dot(ctx_full, wo_s,
                                      preferred_element_type=jnp.float32)

        compute(0, my)

        def step(s, carry):
            wait(mk(t_one, s - 1, s, s - 1, s))

            @pl.when(s < 8)
            def _():
                start(mk(t_one, s, s + 1, s, s + 1))

            @pl.when(s <= 7)
            def _():
                src = jnp.where(s == 1, 0, 8 + s - 1)
                wait(mk(t_zero, src, 8 + s, 8 + s - 1, 8 + s))

                @pl.when(s < 7)
                def _():
                    start(mk(t_zero, 8 + s, 8 + s + 1, 8 + s, 8 + s + 1))

            compute(s, lax.rem(my - s + N_DEV, N_DEV))

            @pl.when(s <= 7)
            def _():
                compute(8 + s, lax.rem(my + s, N_DEV))
            return carry

        lax.fori_loop(1, 9, step, 0)

        out_ref[:, 0:64] = acc_ref[:, 0:64]
        out_ref[:, 192:256] = acc_ref[:, 64:128]
        out_ref[:, 64:192] = acc_ref[:, 128:256]

    return pl.pallas_call(
        body,
        out_shape=jax.ShapeDtypeStruct((B_LOC, SQ, D_MODEL), jnp.float32),
        in_specs=[pl.BlockSpec(memory_space=pltpu.VMEM)] * 5,
        out_specs=pl.BlockSpec(memory_space=pltpu.VMEM),
        scratch_shapes=[
            pltpu.VMEM((N_DEV, D_MODEL, HQ_LOC * DH), jnp.bfloat16),
            pltpu.VMEM((N_DEV, HQ_LOC * DH, D_MODEL), jnp.bfloat16),
            pltpu.SemaphoreType.DMA((N_DEV,)),
            pltpu.SemaphoreType.DMA((N_DEV,)),
            pltpu.SemaphoreType.DMA((N_DEV,)),
            pltpu.SemaphoreType.DMA((N_DEV,)),
            pltpu.VMEM((B_LOC, SQ, D_MODEL), jnp.bfloat16),
            pltpu.VMEM((HQ_LOC * N_DEV, B_LOC, 320, DH), jnp.bfloat16),
            pltpu.VMEM((HQ_LOC * N_DEV, B_LOC, 320, DH), jnp.bfloat16),
            pltpu.VMEM((B_LOC, SQ, D_MODEL), jnp.float32),
        ],
        compiler_params=pltpu.CompilerParams(
            collective_id=0, vmem_limit_bytes=64 * 1024 * 1024),
    )(xb, wq, wo, k_loc, v_loc)


# device time: 97906 ns/iter; 1.0156x vs baseline; 1.0156x over previous
import jax
import jax.numpy as jnp
from jax import lax
from jax.experimental import pallas as pl
from jax.experimental.pallas import tpu as pltpu

N_DEV = 16
B_LOC = 2
SQ = 256
SKV = 256
HQ_LOC = 4
DH = 64
D_MODEL = 512
BLK = 64


def kernel(x, Wq, K_ext, V_ext, Wo):
    i = lax.axis_index("i")
    xb = x.astype(jnp.bfloat16)
    wq = (Wq * 0.125).astype(jnp.bfloat16)
    wo = Wo.astype(jnp.bfloat16)
    k_loc = lax.dynamic_slice_in_dim(K_ext, i * B_LOC, B_LOC, axis=0)
    v_loc = lax.dynamic_slice_in_dim(V_ext, i * B_LOC, B_LOC, axis=0)
    k_loc = jnp.transpose(k_loc, (2, 0, 1, 3)).astype(jnp.bfloat16)
    v_loc = jnp.transpose(v_loc, (2, 0, 1, 3)).astype(jnp.bfloat16)

    def body(x_ref, wq_ref, wo_ref, k_ref, v_ref, out_ref,
             wq_comm, wo_comm, wq_send, wq_recv, wo_send, wo_recv):
        my = lax.axis_index("i")
        right = lax.rem(my + 1, N_DEV)
        left = lax.rem(my + N_DEV - 1, N_DEV)

        barrier = pltpu.get_barrier_semaphore()
        pl.semaphore_signal(barrier, inc=1, device_id=(left,),
                            device_id_type=pl.DeviceIdType.MESH)
        pl.semaphore_signal(barrier, inc=1, device_id=(right,),
                            device_id_type=pl.DeviceIdType.MESH)
        pl.semaphore_wait(barrier, 2)

        qb = lax.broadcasted_iota(jnp.int32, (SQ, SKV), 0) // BLK
        kb = lax.broadcasted_iota(jnp.int32, (SQ, SKV), 1) // BLK
        keep = (qb == kb) | (kb == 0) | (lax.rem(qb + kb, 3) == 0)
        bias = jnp.where(keep, 0.0, -1e9).astype(jnp.float32)

        out_ref[...] = jnp.zeros((B_LOC, SQ, D_MODEL), jnp.float32)
        wq_comm[0] = wq_ref[...]
        wo_comm[0] = wo_ref[...]

        def compute(slot, j):
            wq_s = wq_comm[slot]
            wo_s = wo_comm[slot]
            for b in range(B_LOC):
                q_all = jnp.dot(x_ref[b], wq_s,
                                preferred_element_type=jnp.float32)
                q_all = q_all.astype(jnp.bfloat16)
                ctxs = []
                for hh in range(HQ_LOC):
                    gh = j * HQ_LOC + hh
                    q = q_all[:, hh * DH:(hh + 1) * DH]
                    s = lax.dot_general(
                        q, k_ref[gh, b], (((1,), (1,)), ((), ())),
                        preferred_element_type=jnp.float32) + bias
                    u = jnp.exp(s)
                    r = 1.0 / jnp.sum(u, axis=-1, keepdims=True)
                    ctx = jnp.dot(u.astype(jnp.bfloat16), v_ref[gh, b],
                                  preferred_element_type=jnp.float32) * r
                    ctxs.append(ctx.astype(jnp.bfloat16))
                ctx_full = jnp.concatenate(ctxs, axis=1)
                out_ref[b] += jnp.dot(ctx_full, wo_s,
                                      preferred_element_type=jnp.float32)

        def mk(to_right, src, dst, ssem, rsem):
            tgt = lax.select(to_right, right, left)
            rq = pltpu.make_async_remote_copy(
                src_ref=wq_comm.at[src], dst_ref=wq_comm.at[dst],
                send_sem=wq_send.at[ssem], recv_sem=wq_recv.at[rsem],
                device_id=(tgt,), device_id_type=pl.DeviceIdType.MESH)
            ro = pltpu.make_async_remote_copy(
                src_ref=wo_comm.at[src], dst_ref=wo_comm.at[dst],
                send_sem=wo_send.at[ssem], recv_sem=wo_recv.at[rsem],
                device_id=(tgt,), device_id_type=pl.DeviceIdType.MESH)
            return rq, ro

        def start(descs):
            descs[0].start()
            descs[1].start()

        def wait(descs):
            descs[0].wait()
            descs[1].wait()

        t_one = jnp.bool_(True)
        t_zero = jnp.bool_(False)
        start(mk(t_one, 0, 1, 0, 1))
        start(mk(t_zero, 0, 9, 8, 9))
        compute(0, my)

        def step(s, carry):
            wait(mk(t_one, s - 1, s, s - 1, s))

            @pl.when(s < 8)
            def _():
                start(mk(t_one, s, s + 1, s, s + 1))

            @pl.when(s <= 7)
            def _():
                src = jnp.where(s == 1, 0, 8 + s - 1)
                wait(mk(t_zero, src, 8 + s, 8 + s - 1, 8 + s))

                @pl.when(s < 7)
                def _():
                    start(mk(t_zero, 8 + s, 8 + s + 1, 8 + s, 8 + s + 1))

            compute(s, lax.rem(my - s + N_DEV, N_DEV))

            @pl.when(s <= 7)
            def _():
                compute(8 + s, lax.rem(my + s, N_DEV))
            return carry

        lax.fori_loop(1, 9, step, 0)

    return pl.pallas_call(
        body,
        out_shape=jax.ShapeDtypeStruct((B_LOC, SQ, D_MODEL), jnp.float32),
        in_specs=[pl.BlockSpec(memory_space=pltpu.VMEM)] * 5,
        out_specs=pl.BlockSpec(memory_space=pltpu.VMEM),
        scratch_shapes=[
            pltpu.VMEM((N_DEV, D_MODEL, HQ_LOC * DH), jnp.bfloat16),
            pltpu.VMEM((N_DEV, HQ_LOC * DH, D_MODEL), jnp.bfloat16),
            pltpu.SemaphoreType.DMA((N_DEV,)),
            pltpu.SemaphoreType.DMA((N_DEV,)),
            pltpu.SemaphoreType.DMA((N_DEV,)),
            pltpu.SemaphoreType.DMA((N_DEV,)),
        ],
        compiler_params=pltpu.CompilerParams(
            collective_id=0, vmem_limit_bytes=64 * 1024 * 1024),
    )(xb, wq, wo, k_loc, v_loc)


# device time: 83762 ns/iter; 1.1871x vs baseline; 1.1689x over previous
import jax
import jax.numpy as jnp
from jax import lax
from jax.experimental import pallas as pl
from jax.experimental.pallas import tpu as pltpu

N_DEV = 16
B_LOC = 2
SQ = 256
SKV = 256
HQ_LOC = 4
DH = 64
D_MODEL = 512
BLK = 64


def kernel(x, Wq, K_ext, V_ext, Wo):
    i = lax.axis_index("i")
    xb = x.astype(jnp.bfloat16)
    wq = (Wq * 0.125).astype(jnp.bfloat16)
    wo = Wo.astype(jnp.bfloat16)
    k_loc = lax.dynamic_slice_in_dim(K_ext, i * B_LOC, B_LOC, axis=0)
    v_loc = lax.dynamic_slice_in_dim(V_ext, i * B_LOC, B_LOC, axis=0)
    k_loc = jnp.transpose(k_loc, (2, 0, 1, 3)).astype(jnp.bfloat16)
    v_loc = jnp.transpose(v_loc, (2, 0, 1, 3)).astype(jnp.bfloat16)

    def body(x_ref, wq_ref, wo_ref, k_ref, v_ref, out_ref,
             wq_comm, wo_comm, wq_send, wq_recv, wo_send, wo_recv):
        my = lax.axis_index("i")
        right = lax.rem(my + 1, N_DEV)
        left = lax.rem(my + N_DEV - 1, N_DEV)

        barrier = pltpu.get_barrier_semaphore()
        pl.semaphore_signal(barrier, inc=1, device_id=(left,),
                            device_id_type=pl.DeviceIdType.MESH)
        pl.semaphore_signal(barrier, inc=1, device_id=(right,),
                            device_id_type=pl.DeviceIdType.MESH)
        pl.semaphore_wait(barrier, 2)

        qb = lax.broadcasted_iota(jnp.int32, (SQ, SKV), 0) // BLK
        kb = lax.broadcasted_iota(jnp.int32, (SQ, SKV), 1) // BLK
        keep = (qb == kb) | (kb == 0) | (lax.rem(qb + kb, 3) == 0)
        bias = jnp.where(keep, 0.0, -1e9).astype(jnp.float32)

        out_ref[...] = jnp.zeros((B_LOC, SQ, D_MODEL), jnp.float32)
        wq_comm[0] = wq_ref[...]
        wo_comm[0] = wo_ref[...]

        def compute(slot, j):
            wq_s = wq_comm[slot]
            wo_s = wo_comm[slot]
            for b in range(B_LOC):
                q_all = jnp.dot(x_ref[b], wq_s,
                                preferred_element_type=jnp.float32)
                q_all = q_all.astype(jnp.bfloat16)
                ctxs = []
                for hh in range(HQ_LOC):
                    gh = j * HQ_LOC + hh
                    q = q_all[:, hh * DH:(hh + 1) * DH]
                    s = lax.dot_general(
                        q, k_ref[gh, b], (((1,), (1,)), ((), ())),
                        preferred_element_type=jnp.float32) + bias
                    u = jnp.exp(s)
                    r = 1.0 / jnp.sum(u, axis=-1, keepdims=True)
                    ctx = jnp.dot(u.astype(jnp.bfloat16), v_ref[gh, b],
                                  preferred_element_type=jnp.float32) * r
                    ctxs.append(ctx.astype(jnp.bfloat16))
                ctx_full = jnp.concatenate(ctxs, axis=1)
                out_ref[b] += jnp.dot(ctx_full, wo_s,
                                      preferred_element_type=jnp.float32)

        def mk(to_right, src, dst, ssem, rsem):
            tgt = lax.select(to_right, right, left)
            rq = pltpu.make_async_remote_copy(
                src_ref=wq_comm.at[src], dst_ref=wq_comm.at[dst],
                send_sem=wq_send.at[ssem], recv_sem=wq_recv.at[rsem],
                device_id=(tgt,), device_id_type=pl.DeviceIdType.MESH)
            ro = pltpu.make_async_remote_copy(
                src_ref=wo_comm.at[src], dst_ref=wo_comm.at[dst],
                send_sem=wo_send.at[ssem], recv_sem=wo_recv.at[rsem],
                device_id=(tgt,), device_id_type=pl.DeviceIdType.MESH)
            return rq, ro

        def start(descs):
            descs[0].start()
            descs[1].start()

        def wait(descs):
            descs[0].wait()
            descs[1].wait()

        t_one = jnp.bool_(True)
        t_zero = jnp.bool_(False)
        start(mk(t_one, 0, 1, 0, 1))
        start(mk(t_zero, 0, 9, 8, 9))
        compute(0, my)

        def step(s, carry):
            wait(mk(t_one, s - 1, s, s - 1, s))

            @pl.when(s < 8)
            def _():
                start(mk(t_one, s, s + 1, s, s + 1))

            @pl.when(s <= 7)
            def _():
                src = jnp.where(s == 1, 0, 8 + s - 1)
                wait(mk(t_zero, src, 8 + s, 8 + s - 1, 8 + s))

                @pl.when(s < 7)
                def _():
                    start(mk(t_zero, 8 + s, 8 + s + 1, 8 + s, 8 + s + 1))

            compute(s, lax.rem(my - s + N_DEV, N_DEV))

            @pl.when(s <= 7)
            def _():
                compute(8 + s, lax.rem(my + s, N_DEV))
            return carry

        lax.fori_loop(1, 9, step, 0)

    return pl.pallas_call(
        body,
        out_shape=jax.ShapeDtypeStruct((B_LOC, SQ, D_MODEL), jnp.float32),
        in_specs=[pl.BlockSpec(memory_space=pltpu.VMEM)] * 5,
        out_specs=pl.BlockSpec(memory_space=pltpu.VMEM),
        scratch_shapes=[
            pltpu.VMEM((N_DEV, D_MODEL, HQ_LOC * DH), jnp.bfloat16),
            pltpu.VMEM((N_DEV, HQ_LOC * DH, D_MODEL), jnp.bfloat16),
            pltpu.SemaphoreType.DMA((N_DEV,)),
            pltpu.SemaphoreType.DMA((N_DEV,)),
            pltpu.SemaphoreType.DMA((N_DEV,)),
            pltpu.SemaphoreType.DMA((N_DEV,)),
        ],
        compiler_params=pltpu.CompilerParams(collective_id=0),
    )(xb, wq, wo, k_loc, v_loc)


# device time: 83184 ns/iter; 1.1953x vs baseline; 1.0069x over previous
import jax
import jax.numpy as jnp
from jax import lax
from jax.experimental import pallas as pl
from jax.experimental.pallas import tpu as pltpu

N_DEV = 16
B_LOC = 2
SQ = 256
SKV = 256
HQ_LOC = 4
DH = 64
D_MODEL = 512
BLK = 64


def kernel(x, Wq, K_ext, V_ext, Wo):
    i = lax.axis_index("i")
    xb = x.astype(jnp.bfloat16)
    wq = (Wq * 0.125).astype(jnp.bfloat16)
    wo = Wo.astype(jnp.bfloat16)
    k_loc = lax.dynamic_slice_in_dim(K_ext, i * B_LOC, B_LOC, axis=0)
    v_loc = lax.dynamic_slice_in_dim(V_ext, i * B_LOC, B_LOC, axis=0)
    k_loc = jnp.transpose(k_loc, (2, 0, 1, 3)).astype(jnp.bfloat16)
    v_loc = jnp.transpose(v_loc, (2, 0, 1, 3)).astype(jnp.bfloat16)

    def body(x_ref, wq_ref, wo_ref, k_ref, v_ref, out_ref,
             wq_comm, wo_comm, wq_send, wq_recv, wo_send, wo_recv):
        my = lax.axis_index("i")
        right = lax.rem(my + 1, N_DEV)
        left = lax.rem(my + N_DEV - 1, N_DEV)

        barrier = pltpu.get_barrier_semaphore()
        pl.semaphore_signal(barrier, inc=1, device_id=(left,),
                            device_id_type=pl.DeviceIdType.MESH)
        pl.semaphore_signal(barrier, inc=1, device_id=(right,),
                            device_id_type=pl.DeviceIdType.MESH)
        pl.semaphore_wait(barrier, 2)

        qb = lax.broadcasted_iota(jnp.int32, (SQ, SKV), 0) // BLK
        kb = lax.broadcasted_iota(jnp.int32, (SQ, SKV), 1) // BLK
        keep = (qb == kb) | (kb == 0) | (lax.rem(qb + kb, 3) == 0)
        bias = jnp.where(keep, 0.0, -1e9).astype(jnp.float32)

        out_ref[...] = jnp.zeros((B_LOC, SQ, D_MODEL), jnp.float32)
        wq_comm[0] = wq_ref[...]
        wo_comm[0] = wo_ref[...]

        def compute(slot, j):
            wq_s = wq_comm[slot]
            wo_s = wo_comm[slot]
            x2 = x_ref[...].reshape(B_LOC * SQ, D_MODEL)
            q2 = jnp.dot(x2, wq_s, preferred_element_type=jnp.float32)
            q2 = q2.astype(jnp.bfloat16)
            for b in range(B_LOC):
                q_all = q2[b * SQ:(b + 1) * SQ]
                ctxs = []
                for hh in range(HQ_LOC):
                    gh = j * HQ_LOC + hh
                    q = q_all[:, hh * DH:(hh + 1) * DH]
                    s = lax.dot_general(
                        q, k_ref[gh, b], (((1,), (1,)), ((), ())),
                        preferred_element_type=jnp.float32) + bias
                    u = jnp.exp(s)
                    r = 1.0 / jnp.sum(u, axis=-1, keepdims=True)
                    ctx = jnp.dot(u.astype(jnp.bfloat16), v_ref[gh, b],
                                  preferred_element_type=jnp.float32) * r
                    ctxs.append(ctx.astype(jnp.bfloat16))
                ctx_full = jnp.concatenate(ctxs, axis=1)
                out_ref[b] += jnp.dot(ctx_full, wo_s,
                                      preferred_element_type=jnp.float32)

        def mk(to_right, src, dst, ssem, rsem):
            tgt = lax.select(to_right, right, left)
            rq = pltpu.make_async_remote_copy(
                src_ref=wq_comm.at[src], dst_ref=wq_comm.at[dst],
                send_sem=wq_send.at[ssem], recv_sem=wq_recv.at[rsem],
                device_id=(tgt,), device_id_type=pl.DeviceIdType.MESH)
            ro = pltpu.make_async_remote_copy(
                src_ref=wo_comm.at[src], dst_ref=wo_comm.at[dst],
                send_sem=wo_send.at[ssem], recv_sem=wo_recv.at[rsem],
                device_id=(tgt,), device_id_type=pl.DeviceIdType.MESH)
            return rq, ro

        def start(descs):
            descs[0].start()
            descs[1].start()

        def wait(descs):
            descs[0].wait()
            descs[1].wait()

        t_one = jnp.bool_(True)
        t_zero = jnp.bool_(False)
        start(mk(t_one, 0, 1, 0, 1))
        start(mk(t_zero, 0, 9, 8, 9))
        compute(0, my)

        def step(s, carry):
            wait(mk(t_one, s - 1, s, s - 1, s))

            @pl.when(s < 8)
            def _():
                start(mk(t_one, s, s + 1, s, s + 1))

            @pl.when(s <= 7)
            def _():
                src = jnp.where(s == 1, 0, 8 + s - 1)
                wait(mk(t_zero, src, 8 + s, 8 + s - 1, 8 + s))

                @pl.when(s < 7)
                def _():
                    start(mk(t_zero, 8 + s, 8 + s + 1, 8 + s, 8 + s + 1))

            compute(s, lax.rem(my - s + N_DEV, N_DEV))

            @pl.when(s <= 7)
            def _():
                compute(8 + s, lax.rem(my + s, N_DEV))
            return carry

        lax.fori_loop(1, 9, step, 0)

    return pl.pallas_call(
        body,
        out_shape=jax.ShapeDtypeStruct((B_LOC, SQ, D_MODEL), jnp.float32),
        in_specs=[pl.BlockSpec(memory_space=pltpu.VMEM)] * 5,
        out_specs=pl.BlockSpec(memory_space=pltpu.VMEM),
        scratch_shapes=[
            pltpu.VMEM((N_DEV, D_MODEL, HQ_LOC * DH), jnp.bfloat16),
            pltpu.VMEM((N_DEV, HQ_LOC * DH, D_MODEL), jnp.bfloat16),
            pltpu.SemaphoreType.DMA((N_DEV,)),
            pltpu.SemaphoreType.DMA((N_DEV,)),
            pltpu.SemaphoreType.DMA((N_DEV,)),
            pltpu.SemaphoreType.DMA((N_DEV,)),
        ],
        compiler_params=pltpu.CompilerParams(collective_id=0),
    )(xb, wq, wo, k_loc, v_loc)


# device time: 82125 ns/iter; 1.2107x vs baseline; 1.0129x over previous
import jax
import jax.numpy as jnp
from jax import lax
from jax.experimental import pallas as pl
from jax.experimental.pallas import tpu as pltpu

N_DEV = 16
B_LOC = 2
SQ = 256
SKV = 256
HQ_LOC = 4
DH = 64
D_MODEL = 512
BLK = 64


def kernel(x, Wq, K_ext, V_ext, Wo):
    i = lax.axis_index("i")
    xb = x.astype(jnp.bfloat16)
    wq = (Wq * 0.125).astype(jnp.bfloat16)
    wo = Wo.astype(jnp.bfloat16)
    k_loc = lax.dynamic_slice_in_dim(K_ext, i * B_LOC, B_LOC, axis=0)
    v_loc = lax.dynamic_slice_in_dim(V_ext, i * B_LOC, B_LOC, axis=0)
    k_loc = jnp.transpose(k_loc, (2, 0, 1, 3)).astype(jnp.bfloat16)
    v_loc = jnp.transpose(v_loc, (2, 0, 1, 3)).astype(jnp.bfloat16)

    def body(x_ref, wq_ref, wo_ref, k_ref, v_ref, out_ref,
             wq_comm, wo_comm, wq_send, wq_recv, wo_send, wo_recv):
        my = lax.axis_index("i")
        right = lax.rem(my + 1, N_DEV)
        left = lax.rem(my + N_DEV - 1, N_DEV)

        barrier = pltpu.get_barrier_semaphore()
        pl.semaphore_signal(barrier, inc=1, device_id=(left,),
                            device_id_type=pl.DeviceIdType.MESH)
        pl.semaphore_signal(barrier, inc=1, device_id=(right,),
                            device_id_type=pl.DeviceIdType.MESH)
        pl.semaphore_wait(barrier, 2)

        qb = lax.broadcasted_iota(jnp.int32, (SQ, SKV), 0) // BLK
        kb = lax.broadcasted_iota(jnp.int32, (SQ, SKV), 1) // BLK
        keep = (qb == kb) | (kb == 0) | (lax.rem(qb + kb, 3) == 0)
        bias = jnp.where(keep, 0.0, -1e9).astype(jnp.float32)

        out_ref[...] = jnp.zeros((B_LOC, SQ, D_MODEL), jnp.float32)
        wq_comm[0] = wq_ref[...]
        wo_comm[0] = wo_ref[...]

        def compute(slot, j):
            wq_s = wq_comm[slot]
            wo_s = wo_comm[slot]
            x2 = x_ref[...].reshape(B_LOC * SQ, D_MODEL)
            q2 = jnp.dot(x2, wq_s, preferred_element_type=jnp.float32)
            q2 = q2.astype(jnp.bfloat16)
            ctx_rows = []
            for b in range(B_LOC):
                q_all = q2[b * SQ:(b + 1) * SQ]
                ctxs = []
                for hh in range(HQ_LOC):
                    gh = j * HQ_LOC + hh
                    q = q_all[:, hh * DH:(hh + 1) * DH]
                    s = lax.dot_general(
                        q, k_ref[gh, b], (((1,), (1,)), ((), ())),
                        preferred_element_type=jnp.float32) + bias
                    u = jnp.exp(s)
                    r = 1.0 / jnp.sum(u, axis=-1, keepdims=True)
                    ctx = jnp.dot(u.astype(jnp.bfloat16), v_ref[gh, b],
                                  preferred_element_type=jnp.float32) * r
                    ctxs.append(ctx.astype(jnp.bfloat16))
                ctx_rows.append(jnp.concatenate(ctxs, axis=1))
            ctx2 = jnp.concatenate(ctx_rows, axis=0)
            o2 = jnp.dot(ctx2, wo_s, preferred_element_type=jnp.float32)
            out_ref[...] += o2.reshape(B_LOC, SQ, D_MODEL)

        def mk(to_right, src, dst, ssem, rsem):
            tgt = lax.select(to_right, right, left)
            rq = pltpu.make_async_remote_copy(
                src_ref=wq_comm.at[src], dst_ref=wq_comm.at[dst],
                send_sem=wq_send.at[ssem], recv_sem=wq_recv.at[rsem],
                device_id=(tgt,), device_id_type=pl.DeviceIdType.MESH)
            ro = pltpu.make_async_remote_copy(
                src_ref=wo_comm.at[src], dst_ref=wo_comm.at[dst],
                send_sem=wo_send.at[ssem], recv_sem=wo_recv.at[rsem],
                device_id=(tgt,), device_id_type=pl.DeviceIdType.MESH)
            return rq, ro

        def start(descs):
            descs[0].start()
            descs[1].start()

        def wait(descs):
            descs[0].wait()
            descs[1].wait()

        t_one = jnp.bool_(True)
        t_zero = jnp.bool_(False)
        start(mk(t_one, 0, 1, 0, 1))
        start(mk(t_zero, 0, 9, 8, 9))
        compute(0, my)

        def step(s, carry):
            wait(mk(t_one, s - 1, s, s - 1, s))

            @pl.when(s < 8)
            def _():
                start(mk(t_one, s, s + 1, s, s + 1))

            @pl.when(s <= 7)
            def _():
                src = jnp.where(s == 1, 0, 8 + s - 1)
                wait(mk(t_zero, src, 8 + s, 8 + s - 1, 8 + s))

                @pl.when(s < 7)
                def _():
                    start(mk(t_zero, 8 + s, 8 + s + 1, 8 + s, 8 + s + 1))

            compute(s, lax.rem(my - s + N_DEV, N_DEV))

            @pl.when(s <= 7)
            def _():
                compute(8 + s, lax.rem(my + s, N_DEV))
            return carry

        lax.fori_loop(1, 9, step, 0)

    return pl.pallas_call(
        body,
        out_shape=jax.ShapeDtypeStruct((B_LOC, SQ, D_MODEL), jnp.float32),
        in_specs=[pl.BlockSpec(memory_space=pltpu.VMEM)] * 5,
        out_specs=pl.BlockSpec(memory_space=pltpu.VMEM),
        scratch_shapes=[
            pltpu.VMEM((N_DEV, D_MODEL, HQ_LOC * DH), jnp.bfloat16),
            pltpu.VMEM((N_DEV, HQ_LOC * DH, D_MODEL), jnp.bfloat16),
            pltpu.SemaphoreType.DMA((N_DEV,)),
            pltpu.SemaphoreType.DMA((N_DEV,)),
            pltpu.SemaphoreType.DMA((N_DEV,)),
            pltpu.SemaphoreType.DMA((N_DEV,)),
        ],
        compiler_params=pltpu.CompilerParams(collective_id=0),
    )(xb, wq, wo, k_loc, v_loc)
